# baseline (speedup 1.0000x reference)
import functools

import jax
import jax.numpy as jnp
from jax.experimental import pallas as pl
from jax.experimental.pallas import tpu as pltpu

BB = 32
SUB = 8


def _f32dot(a, b, dims):
    return jax.lax.dot_general(a, b, dims, preferred_element_type=jnp.float32)


def _ln(x, g, b, eps=1e-5):
    m = jnp.mean(x, axis=-1, keepdims=True)
    v = jnp.mean((x - m) ** 2, axis=-1, keepdims=True)
    return (x - m) / jnp.sqrt(v + eps) * g + b


def _block_kernel(xl_ref, xr_ref, Wq_ref, bq_ref, Wk_ref, bk_ref, Wv_ref,
                  bv_ref, Wo_ref, bo_ref, agl_ref, abl_ref, agr_ref, abr_ref,
                  centers_ref, Wrp_ref, brp_ref, WeBD_ref, beBD_ref,
                  mgl_ref, mbl_ref, mgr_ref, mbr_ref, yl_ref, yr_ref,
                  *, bb, C, T):
    for s in range(bb // SUB):
        _sub_block(xl_ref, xr_ref, Wq_ref, bq_ref, Wk_ref, bk_ref, Wv_ref,
                   bv_ref, Wo_ref, bo_ref, agl_ref, abl_ref, agr_ref, abr_ref,
                   centers_ref, Wrp_ref, brp_ref, WeBD_ref, beBD_ref,
                   mgl_ref, mbl_ref, mgr_ref, mbr_ref, yl_ref, yr_ref,
                   s, SUB, C, T)


def _sub_block(xl_ref, xr_ref, Wq_ref, bq_ref, Wk_ref, bk_ref, Wv_ref,
               bv_ref, Wo_ref, bo_ref, agl_ref, abl_ref, agr_ref, abr_ref,
               centers_ref, Wrp_ref, brp_ref, WeBD_ref, beBD_ref,
               mgl_ref, mbl_ref, mgr_ref, mbr_ref, yl_ref, yr_ref,
               s, bb, C, T):
    NT = bb * C
    sl = slice(s * bb, (s + 1) * bb)
    xl = xl_ref[sl].reshape(NT, T)
    xr = xr_ref[sl].reshape(NT, T)

    q = (_f32dot(xl, Wq_ref[...], (((1,), (1,)), ((), ()))) + bq_ref[...]).reshape(bb, C, T)
    k = (_f32dot(xr, Wk_ref[...], (((1,), (1,)), ((), ()))) + bk_ref[...]).reshape(bb, C, T)
    v = (_f32dot(xl - xr, Wv_ref[...], (((1,), (1,)), ((), ()))) + bv_ref[...]).reshape(bb, C, T)
    energy = _f32dot(q, k, (((2,), (2,)), ((0,), (0,)))) * (1.0 / (T ** 0.5))
    energy = energy - jnp.max(energy, axis=-1, keepdims=True)
    ex = jnp.exp(energy)
    attn = ex / jnp.sum(ex, axis=-1, keepdims=True)
    ol = _f32dot(attn, v, (((2,), (1,)), ((0,), (0,)))).reshape(NT, T)
    orr = _f32dot(attn, v, (((1,), (1,)), ((0,), (0,)))).reshape(NT, T)
    ol = _f32dot(ol, Wo_ref[...], (((1,), (1,)), ((), ()))) + bo_ref[...]
    orr = _f32dot(orr, Wo_ref[...], (((1,), (1,)), ((), ()))) + bo_ref[...]
    out_l = _ln(ol, agl_ref[...], abl_ref[...]) + xl
    out_r = _ln(orr, agr_ref[...], abr_ref[...]) + xr

    X = jnp.concatenate([out_l, out_r], axis=1)
    xp = _f32dot(X, Wrp_ref[...], (((1,), (1,)), ((), ()))) + brp_ref[...]
    xp = xp / jnp.maximum(jnp.sqrt(jnp.sum(xp * xp, axis=1, keepdims=True)), 1e-12)
    cen = centers_ref[...]
    cen = cen / jnp.maximum(jnp.sqrt(jnp.sum(cen * cen, axis=1, keepdims=True)), 1e-12)
    sim = _f32dot(xp, cen, (((1,), (1,)), ((), ())))

    lane = jax.lax.broadcasted_iota(jnp.int32, (NT, C), 1)
    m1 = jnp.max(sim, axis=1, keepdims=True)
    i1 = jnp.min(jnp.where(sim == m1, lane, C), axis=1, keepdims=True)
    z1 = lane == i1
    sim2 = jnp.where(z1, -1e30, sim)
    m2 = jnp.max(sim2, axis=1, keepdims=True)
    i2 = jnp.min(jnp.where(sim2 == m2, lane, C), axis=1, keepdims=True)
    Z = (z1 | (lane == i2)).astype(jnp.float32)

    acc = _f32dot(Z, beBD_ref[...], (((1,), (0,)), ((), ())))
    Xb = X.astype(jnp.bfloat16)
    for e in range(C):
        Ye = _f32dot(Xb, WeBD_ref[e], (((1,), (1,)), ((), ())))
        acc = acc + Ye * Z[:, e][:, None]

    accl, accr = acc[:, :T], acc[:, T:]
    yl = _ln(accl, mgl_ref[...], mbl_ref[...]) + out_l
    yr = _ln(accr, mgr_ref[...], mbr_ref[...]) + out_r
    yl_ref[sl] = yl.reshape(bb, C, T)
    yr_ref[sl] = yr.reshape(bb, C, T)


def kernel(x_l, x_r, Wq, bq, Wk, bk, Wv, bv, Wo, bo, a_gl, a_bl, a_gr, a_br,
           centers, Wrp, brp, We, be, m_gl, m_bl, m_gr, m_br):
    B, C, T = x_l.shape
    WeBD = jnp.zeros((C, 2 * T, 2 * T), jnp.bfloat16)
    We16 = We.astype(jnp.bfloat16)
    WeBD = WeBD.at[:, :T, :T].set(We16).at[:, T:, T:].set(We16)
    beBD = jnp.concatenate([be, be], axis=1)

    full = lambda *shape: pl.BlockSpec(shape, lambda i: (0,) * len(shape))
    blk = pl.BlockSpec((BB, C, T), lambda i: (i, 0, 0))

    f = pl.pallas_call(
        functools.partial(_block_kernel, bb=BB, C=C, T=T),
        grid=(B // BB,),
        in_specs=[
            blk, blk,
            full(T, T), full(T), full(T, T), full(T), full(T, T), full(T),
            full(T, T), full(T), full(T), full(T), full(T), full(T),
            full(C, 32), full(32, 2 * T), full(32),
            full(C, 2 * T, 2 * T), full(C, 2 * T),
            full(T), full(T), full(T), full(T),
        ],
        out_specs=[blk, blk],
        out_shape=[jax.ShapeDtypeStruct((B, C, T), jnp.float32)] * 2,
        compiler_params=pltpu.CompilerParams(
            dimension_semantics=("arbitrary",),
        ),
    )
    yl, yr = f(x_l, x_r, Wq, bq, Wk, bk, Wv, bv, Wo, bo, a_gl, a_bl, a_gr,
               a_br, centers, Wrp, brp, WeBD, beBD, m_gl, m_bl, m_gr, m_br)
    return (yl, yr)


# speedup vs baseline: 1.2826x; 1.2826x over previous
import functools

import jax
import jax.numpy as jnp
from jax.experimental import pallas as pl
from jax.experimental.pallas import tpu as pltpu

BB = 32
SUB = 32


def _f32dot(a, b, dims):
    return jax.lax.dot_general(a, b, dims, preferred_element_type=jnp.float32)


def _ln(x, g, b, eps=1e-5):
    m = jnp.mean(x, axis=-1, keepdims=True)
    v = jnp.mean((x - m) ** 2, axis=-1, keepdims=True)
    return (x - m) / jnp.sqrt(v + eps) * g + b


def _block_kernel(xl_ref, xr_ref, Wq_ref, bq_ref, Wk_ref, bk_ref, Wv_ref,
                  bv_ref, Wo_ref, bo_ref, agl_ref, abl_ref, agr_ref, abr_ref,
                  centers_ref, Wrp_ref, brp_ref, WeBD_ref, beBD_ref,
                  mgl_ref, mbl_ref, mgr_ref, mbr_ref, yl_ref, yr_ref,
                  *, bb, C, T):
    for s in range(bb // SUB):
        _sub_block(xl_ref, xr_ref, Wq_ref, bq_ref, Wk_ref, bk_ref, Wv_ref,
                   bv_ref, Wo_ref, bo_ref, agl_ref, abl_ref, agr_ref, abr_ref,
                   centers_ref, Wrp_ref, brp_ref, WeBD_ref, beBD_ref,
                   mgl_ref, mbl_ref, mgr_ref, mbr_ref, yl_ref, yr_ref,
                   s, SUB, C, T)


def _sub_block(xl_ref, xr_ref, Wq_ref, bq_ref, Wk_ref, bk_ref, Wv_ref,
               bv_ref, Wo_ref, bo_ref, agl_ref, abl_ref, agr_ref, abr_ref,
               centers_ref, Wrp_ref, brp_ref, WeBD_ref, beBD_ref,
               mgl_ref, mbl_ref, mgr_ref, mbr_ref, yl_ref, yr_ref,
               s, bb, C, T):
    NT = bb * C
    sl = slice(s * bb, (s + 1) * bb)
    xl = xl_ref[sl].reshape(NT, T)
    xr = xr_ref[sl].reshape(NT, T)

    q = (_f32dot(xl, Wq_ref[...], (((1,), (1,)), ((), ()))) + bq_ref[...]).reshape(bb, C, T)
    k = (_f32dot(xr, Wk_ref[...], (((1,), (1,)), ((), ()))) + bk_ref[...]).reshape(bb, C, T)
    v = (_f32dot(xl - xr, Wv_ref[...], (((1,), (1,)), ((), ()))) + bv_ref[...]).reshape(bb, C, T)
    energy = _f32dot(q, k, (((2,), (2,)), ((0,), (0,)))) * (1.0 / (T ** 0.5))
    energy = energy - jnp.max(energy, axis=-1, keepdims=True)
    ex = jnp.exp(energy)
    attn = ex / jnp.sum(ex, axis=-1, keepdims=True)
    ol = _f32dot(attn, v, (((2,), (1,)), ((0,), (0,)))).reshape(NT, T)
    orr = _f32dot(attn, v, (((1,), (1,)), ((0,), (0,)))).reshape(NT, T)
    ol = _f32dot(ol, Wo_ref[...], (((1,), (1,)), ((), ()))) + bo_ref[...]
    orr = _f32dot(orr, Wo_ref[...], (((1,), (1,)), ((), ()))) + bo_ref[...]
    out_l = _ln(ol, agl_ref[...], abl_ref[...]) + xl
    out_r = _ln(orr, agr_ref[...], abr_ref[...]) + xr

    X = jnp.concatenate([out_l, out_r], axis=1)
    xp = _f32dot(X, Wrp_ref[...], (((1,), (1,)), ((), ()))) + brp_ref[...]
    xp = xp / jnp.maximum(jnp.sqrt(jnp.sum(xp * xp, axis=1, keepdims=True)), 1e-12)
    cen = centers_ref[...]
    cen = cen / jnp.maximum(jnp.sqrt(jnp.sum(cen * cen, axis=1, keepdims=True)), 1e-12)
    sim = _f32dot(xp, cen, (((1,), (1,)), ((), ())))

    lane = jax.lax.broadcasted_iota(jnp.int32, (NT, C), 1)
    m1 = jnp.max(sim, axis=1, keepdims=True)
    i1 = jnp.min(jnp.where(sim == m1, lane, C), axis=1, keepdims=True)
    z1 = lane == i1
    sim2 = jnp.where(z1, -1e30, sim)
    m2 = jnp.max(sim2, axis=1, keepdims=True)
    i2 = jnp.min(jnp.where(sim2 == m2, lane, C), axis=1, keepdims=True)
    Z = (z1 | (lane == i2)).astype(jnp.float32)

    bsum = _f32dot(Z, beBD_ref[...], (((1,), (0,)), ((), ())))
    Xb = X.astype(jnp.bfloat16)
    Zb = Z.astype(jnp.bfloat16)
    accb = None
    for e in range(C):
        Ye = _f32dot(Xb, WeBD_ref[e], (((1,), (1,)), ((), ()))).astype(jnp.bfloat16)
        t = Ye * Zb[:, e][:, None]
        accb = t if accb is None else accb + t
    acc = accb.astype(jnp.float32) + bsum

    accl, accr = acc[:, :T], acc[:, T:]
    yl = _ln(accl, mgl_ref[...], mbl_ref[...]) + out_l
    yr = _ln(accr, mgr_ref[...], mbr_ref[...]) + out_r
    yl_ref[sl] = yl.reshape(bb, C, T)
    yr_ref[sl] = yr.reshape(bb, C, T)


def kernel(x_l, x_r, Wq, bq, Wk, bk, Wv, bv, Wo, bo, a_gl, a_bl, a_gr, a_br,
           centers, Wrp, brp, We, be, m_gl, m_bl, m_gr, m_br):
    B, C, T = x_l.shape
    WeBD = jnp.zeros((C, 2 * T, 2 * T), jnp.bfloat16)
    We16 = We.astype(jnp.bfloat16)
    WeBD = WeBD.at[:, :T, :T].set(We16).at[:, T:, T:].set(We16)
    beBD = jnp.concatenate([be, be], axis=1)

    full = lambda *shape: pl.BlockSpec(shape, lambda i: (0,) * len(shape))
    blk = pl.BlockSpec((BB, C, T), lambda i: (i, 0, 0))

    f = pl.pallas_call(
        functools.partial(_block_kernel, bb=BB, C=C, T=T),
        grid=(B // BB,),
        in_specs=[
            blk, blk,
            full(T, T), full(T), full(T, T), full(T), full(T, T), full(T),
            full(T, T), full(T), full(T), full(T), full(T), full(T),
            full(C, 32), full(32, 2 * T), full(32),
            full(C, 2 * T, 2 * T), full(C, 2 * T),
            full(T), full(T), full(T), full(T),
        ],
        out_specs=[blk, blk],
        out_shape=[jax.ShapeDtypeStruct((B, C, T), jnp.float32)] * 2,
        compiler_params=pltpu.CompilerParams(
            dimension_semantics=("arbitrary",),
        ),
    )
    yl, yr = f(x_l, x_r, Wq, bq, Wk, bk, Wv, bv, Wo, bo, a_gl, a_bl, a_gr,
               a_br, centers, Wrp, brp, WeBD, beBD, m_gl, m_bl, m_gr, m_br)
    return (yl, yr)
